# Initial kernel scaffold; baseline (speedup 1.0000x reference)
import jax
import jax.numpy as jnp
from jax.experimental import pallas as pl


def kernel(input, input_lengths, emb_weight, lin_weight):
    raise NotImplementedError("write your pallas kernel here")


# speedup vs baseline: 2.8420x; 2.8420x over previous
import functools

import jax
import jax.numpy as jnp
from jax import lax
from jax.experimental import pallas as pl
from jax.experimental.pallas import tpu as pltpu
from jax.experimental.pallas import tpu_sc as plsc


_ROWS_BLK = 8192


def _tvec_body(emb_ref, w_ref, t_ref):
    i = pl.program_id(0)
    e = emb_ref[...]
    w = w_ref[...]
    row = lax.broadcasted_iota(jnp.int32, e.shape, 0) + i * _ROWS_BLK
    e = jnp.where(row == 0, 0.0, e)
    t_ref[...] = jnp.sum(e * w, axis=1)


def _compute_tvec(emb_weight, lin_weight):
    V, D = emb_weight.shape
    nblk = pl.cdiv(V, _ROWS_BLK)
    return pl.pallas_call(
        _tvec_body,
        grid=(nblk,),
        in_specs=[
            pl.BlockSpec((_ROWS_BLK, D), lambda i: (i, 0)),
            pl.BlockSpec((1, D), lambda i: (0, 0)),
        ],
        out_specs=pl.BlockSpec((_ROWS_BLK,), lambda i: (i,)),
        out_shape=jax.ShapeDtypeStruct((nblk * _ROWS_BLK,), jnp.float32),
    )(emb_weight, lin_weight)


@functools.cache
def _make_sc_lookup(B, L, TV):
    info = plsc.get_sparse_core_info()
    NC, NS = info.num_cores, info.num_subcores
    NW = NC * NS
    TB = B // NW
    LCH = 40
    NCH = L // LCH
    NJ = TB // 128
    assert B % NW == 0 and L % LCH == 0 and TB % 128 == 0

    mesh = plsc.VectorSubcoreMesh(core_axis_name="c", subcore_axis_name="s")

    @functools.partial(
        pl.kernel,
        out_type=jax.ShapeDtypeStruct((B,), jnp.float32),
        mesh=mesh,
        scratch_types=[
            pltpu.VMEM((LCH, TB), jnp.int32),
            pltpu.VMEM((LCH, TB), jnp.float32),
            pltpu.VMEM((TB,), jnp.float32),
            pltpu.SemaphoreType.DMA,
        ],
    )
    def sc_lookup(idxT_hbm, t_hbm, out_hbm, idx_v, vals_v, acc_v, sem):
        wid = lax.axis_index("s") * NC + lax.axis_index("c")
        base = wid * TB

        zero = jnp.zeros((16,), jnp.float32)
        for g in range(TB // 16):
            acc_v[pl.ds(g * 16, 16)] = zero

        def chunk_body(c, carry):
            pltpu.sync_copy(
                idxT_hbm.at[pl.ds(c * LCH, LCH), pl.ds(base, TB)], idx_v
            )

            def row_body(i, carry):
                descs = [
                    pltpu.async_copy(
                        t_hbm.at[idx_v.at[i, pl.ds(j * 128, 128)]],
                        vals_v.at[i, pl.ds(j * 128, 128)],
                        sem,
                    )
                    for j in range(NJ)
                ]
                for d in descs:
                    d.wait()
                return carry

            lax.fori_loop(0, LCH, row_body, 0)

            def red_body(g, carry):
                off = pl.multiple_of(g * 16, 16)
                s = acc_v[pl.ds(off, 16)]

                def l_body(l, s):
                    return s + vals_v[l, pl.ds(off, 16)]

                s = lax.fori_loop(0, LCH, l_body, s)
                acc_v[pl.ds(off, 16)] = s
                return carry

            lax.fori_loop(0, TB // 16, red_body, 0)
            return carry

        lax.fori_loop(0, NCH, chunk_body, 0)
        pltpu.sync_copy(acc_v, out_hbm.at[pl.ds(base, TB)])

    return sc_lookup


def kernel(input, input_lengths, emb_weight, lin_weight):
    B, L = input.shape
    tvec = _compute_tvec(emb_weight, lin_weight)
    idxT = input.T
    out = _make_sc_lookup(B, L, tvec.shape[0])(idxT, tvec)
    return out.reshape(B, 1)


# speedup vs baseline: 3.2868x; 1.1565x over previous
import functools

import jax
import jax.numpy as jnp
from jax import lax
from jax.experimental import pallas as pl
from jax.experimental.pallas import tpu as pltpu
from jax.experimental.pallas import tpu_sc as plsc


_ROWS_BLK = 8192


def _tvec_body(emb_ref, w_ref, t_ref):
    i = pl.program_id(0)
    e = emb_ref[...]
    w = w_ref[...]
    row = lax.broadcasted_iota(jnp.int32, e.shape, 0) + i * _ROWS_BLK
    e = jnp.where(row == 0, 0.0, e)
    t_ref[...] = jnp.sum(e * w, axis=1)


def _compute_tvec(emb_weight, lin_weight):
    V, D = emb_weight.shape
    nblk = pl.cdiv(V, _ROWS_BLK)
    return pl.pallas_call(
        _tvec_body,
        grid=(nblk,),
        in_specs=[
            pl.BlockSpec((_ROWS_BLK, D), lambda i: (i, 0)),
            pl.BlockSpec((1, D), lambda i: (0, 0)),
        ],
        out_specs=pl.BlockSpec((_ROWS_BLK,), lambda i: (i,)),
        out_shape=jax.ShapeDtypeStruct((nblk * _ROWS_BLK,), jnp.float32),
    )(emb_weight, lin_weight)


@functools.cache
def _make_sc_lookup(B, L, TV):
    info = plsc.get_sparse_core_info()
    NC, NS = info.num_cores, info.num_subcores
    NW = NC * NS
    TB = B // NW
    LCH = 40
    NCH = L // LCH
    QR = 10
    NQ = LCH // QR
    NJ = TB // 128
    NG = TB // 16
    assert B % NW == 0 and L % LCH == 0 and TB % 128 == 0 and LCH % QR == 0

    mesh = plsc.VectorSubcoreMesh(core_axis_name="c", subcore_axis_name="s")

    @functools.partial(
        pl.kernel,
        out_type=jax.ShapeDtypeStruct((B,), jnp.float32),
        mesh=mesh,
        scratch_types=[
            pltpu.VMEM((LCH, TB), jnp.int32),
            pltpu.VMEM((LCH, TB), jnp.float32),
            pltpu.VMEM((TB,), jnp.float32),
            pltpu.SemaphoreType.DMA,
            pltpu.SemaphoreType.DMA,
        ],
    )
    def sc_lookup(idxT_hbm, t_hbm, out_hbm, idx_v, vals_v, acc_v, sem0, sem1):
        wid = lax.axis_index("s") * NC + lax.axis_index("c")
        base = wid * TB
        sems = (sem0, sem1)

        zero = jnp.zeros((16,), jnp.float32)
        for g in range(NG):
            acc_v[pl.ds(g * 16, 16)] = zero

        def q_descs(q, sem):
            return [
                pltpu.make_async_copy(
                    t_hbm.at[idx_v.at[q * QR + r, pl.ds(j * 128, 128)]],
                    vals_v.at[q * QR + r, pl.ds(j * 128, 128)],
                    sem,
                )
                for r in range(QR)
                for j in range(NJ)
            ]

        def reduce_quarter(q):
            def red_body(g, carry):
                off = pl.multiple_of(g * 16, 16)
                s = acc_v[pl.ds(off, 16)]
                for r in range(QR):
                    s = s + vals_v[q * QR + r, pl.ds(off, 16)]
                acc_v[pl.ds(off, 16)] = s
                return carry

            lax.fori_loop(0, NG, red_body, 0)

        def chunk_body(c, carry):
            pltpu.sync_copy(
                idxT_hbm.at[pl.ds(c * LCH, LCH), pl.ds(base, TB)], idx_v
            )
            for d in q_descs(0, sems[0]):
                d.start()
            for d in q_descs(1, sems[1]):
                d.start()
            for q in range(NQ):
                for d in q_descs(q, sems[q % 2]):
                    d.wait()
                if q + 2 < NQ:
                    for d in q_descs(q + 2, sems[q % 2]):
                        d.start()
                reduce_quarter(q)
            return carry

        lax.fori_loop(0, NCH, chunk_body, 0)
        pltpu.sync_copy(acc_v, out_hbm.at[pl.ds(base, TB)])

    return sc_lookup


def kernel(input, input_lengths, emb_weight, lin_weight):
    B, L = input.shape
    tvec = _compute_tvec(emb_weight, lin_weight)
    idxT = input.T
    out = _make_sc_lookup(B, L, tvec.shape[0])(idxT, tvec)
    return out.reshape(B, 1)


# speedup vs baseline: 4.8092x; 1.4632x over previous
import functools

import jax
import jax.numpy as jnp
from jax import lax
from jax.experimental import pallas as pl
from jax.experimental.pallas import tpu as pltpu
from jax.experimental.pallas import tpu_sc as plsc


_ROWS_BLK = 8192


def _tvec_body(emb_ref, w_ref, t_ref):
    i = pl.program_id(0)
    e = emb_ref[...]
    w = w_ref[...]
    t = lax.dot_general(
        w, e, (((1,), (1,)), ((), ())), preferred_element_type=jnp.float32
    )
    col = lax.broadcasted_iota(jnp.int32, t.shape, 1) + i * _ROWS_BLK
    t_ref[...] = jnp.where(col == 0, 0.0, t)[None]


def _compute_tvec(emb_weight, lin_weight):
    V, D = emb_weight.shape
    nblk = pl.cdiv(V, _ROWS_BLK)
    out = pl.pallas_call(
        _tvec_body,
        grid=(nblk,),
        in_specs=[
            pl.BlockSpec((_ROWS_BLK, D), lambda i: (i, 0)),
            pl.BlockSpec((1, D), lambda i: (0, 0)),
        ],
        out_specs=pl.BlockSpec((1, 1, _ROWS_BLK), lambda i: (i, 0, 0)),
        out_shape=jax.ShapeDtypeStruct((nblk, 1, _ROWS_BLK), jnp.float32),
    )(emb_weight, lin_weight)
    return out.reshape(-1)


@functools.cache
def _make_sc_lookup(B, L, TV):
    info = plsc.get_sparse_core_info()
    NC, NS = info.num_cores, info.num_subcores
    NW = NC * NS
    TB = B // NW
    LCH = 40
    NCH = L // LCH
    QR = 10
    NQ = LCH // QR
    NJ = TB // 128
    NG = TB // 16
    assert B % NW == 0 and L % LCH == 0 and TB % 128 == 0 and LCH % QR == 0

    mesh = plsc.VectorSubcoreMesh(core_axis_name="c", subcore_axis_name="s")

    @functools.partial(
        pl.kernel,
        out_type=jax.ShapeDtypeStruct((B,), jnp.float32),
        mesh=mesh,
        scratch_types=[
            pltpu.VMEM((LCH, TB), jnp.int32),
            pltpu.VMEM((LCH, TB), jnp.float32),
            pltpu.VMEM((TB,), jnp.float32),
            pltpu.SemaphoreType.DMA,
            pltpu.SemaphoreType.DMA,
        ],
    )
    def sc_lookup(idxT_hbm, t_hbm, out_hbm, idx_v, vals_v, acc_v, sem0, sem1):
        wid = lax.axis_index("s") * NC + lax.axis_index("c")
        base = wid * TB
        sems = (sem0, sem1)

        zero = jnp.zeros((16,), jnp.float32)
        for g in range(NG):
            acc_v[pl.ds(g * 16, 16)] = zero

        def q_descs(q, sem):
            return [
                pltpu.make_async_copy(
                    t_hbm.at[idx_v.at[q * QR + r, pl.ds(j * 128, 128)]],
                    vals_v.at[q * QR + r, pl.ds(j * 128, 128)],
                    sem,
                )
                for r in range(QR)
                for j in range(NJ)
            ]

        def reduce_quarter(q):
            def red_body(g, carry):
                off = pl.multiple_of(g * 16, 16)
                s = acc_v[pl.ds(off, 16)]
                for r in range(QR):
                    s = s + vals_v[q * QR + r, pl.ds(off, 16)]
                acc_v[pl.ds(off, 16)] = s
                return carry

            lax.fori_loop(0, NG, red_body, 0)

        def chunk_body(c, carry):
            pltpu.sync_copy(
                idxT_hbm.at[pl.ds(c * LCH, LCH), pl.ds(base, TB)], idx_v
            )
            for d in q_descs(0, sems[0]):
                d.start()
            for d in q_descs(1, sems[1]):
                d.start()
            for q in range(NQ):
                for d in q_descs(q, sems[q % 2]):
                    d.wait()
                if q + 2 < NQ:
                    for d in q_descs(q + 2, sems[q % 2]):
                        d.start()
                reduce_quarter(q)
            return carry

        lax.fori_loop(0, NCH, chunk_body, 0)
        pltpu.sync_copy(acc_v, out_hbm.at[pl.ds(base, TB)])

    return sc_lookup


def kernel(input, input_lengths, emb_weight, lin_weight):
    B, L = input.shape
    tvec = _compute_tvec(emb_weight, lin_weight)
    idxT = input.T
    out = _make_sc_lookup(B, L, tvec.shape[0])(idxT, tvec)
    return out.reshape(B, 1)


# speedup vs baseline: 4.8533x; 1.0092x over previous
import functools

import jax
import jax.numpy as jnp
from jax import lax
from jax.experimental import pallas as pl
from jax.experimental.pallas import tpu as pltpu
from jax.experimental.pallas import tpu_sc as plsc


_ROWS_BLK = 8192


def _tvec_body(emb_ref, w_ref, t_ref):
    i = pl.program_id(0)
    e = emb_ref[...]
    w = w_ref[...]
    t = lax.dot_general(
        w, e, (((1,), (1,)), ((), ())), preferred_element_type=jnp.float32
    )
    col = lax.broadcasted_iota(jnp.int32, t.shape, 1) + i * _ROWS_BLK
    t_ref[...] = jnp.where(col == 0, 0.0, t)[None]


def _compute_tvec(emb_weight, lin_weight):
    V, D = emb_weight.shape
    nblk = pl.cdiv(V, _ROWS_BLK)
    out = pl.pallas_call(
        _tvec_body,
        grid=(nblk,),
        in_specs=[
            pl.BlockSpec((_ROWS_BLK, D), lambda i: (i, 0)),
            pl.BlockSpec((1, D), lambda i: (0, 0)),
        ],
        out_specs=pl.BlockSpec((1, 1, _ROWS_BLK), lambda i: (i, 0, 0)),
        out_shape=jax.ShapeDtypeStruct((nblk, 1, _ROWS_BLK), jnp.float32),
    )(emb_weight, lin_weight)
    return out.reshape(-1)


_TRB = 512


def _transpose_body(x_ref, o_ref):
    o_ref[...] = x_ref[...].T


def _transpose_idx(x):
    B, L = x.shape
    return pl.pallas_call(
        _transpose_body,
        grid=(B // _TRB,),
        in_specs=[pl.BlockSpec((_TRB, L), lambda i: (i, 0))],
        out_specs=pl.BlockSpec((L, _TRB), lambda i: (0, i)),
        out_shape=jax.ShapeDtypeStruct((L, B), jnp.int32),
    )(x)


@functools.cache
def _make_sc_lookup(B, L, TV):
    info = plsc.get_sparse_core_info()
    NC, NS = info.num_cores, info.num_subcores
    NW = NC * NS
    TB = B // NW
    LCH = 40
    NCH = L // LCH
    QR = 10
    NQ = LCH // QR
    NJ = TB // 128
    NG = TB // 16
    assert B % NW == 0 and L % LCH == 0 and TB % 128 == 0 and LCH % QR == 0

    mesh = plsc.VectorSubcoreMesh(core_axis_name="c", subcore_axis_name="s")

    @functools.partial(
        pl.kernel,
        out_type=jax.ShapeDtypeStruct((B,), jnp.float32),
        mesh=mesh,
        scratch_types=[
            pltpu.VMEM((LCH, TB), jnp.int32),
            pltpu.VMEM((LCH, TB), jnp.float32),
            pltpu.VMEM((TB,), jnp.float32),
            pltpu.VMEM_SHARED((TV,), jnp.float32),
            pltpu.SemaphoreType.DMA,
            pltpu.SemaphoreType.DMA,
        ],
    )
    def sc_lookup(idxT_hbm, t_hbm, out_hbm, idx_v, vals_v, acc_v, t_sh, sem0, sem1):
        wid = lax.axis_index("s") * NC + lax.axis_index("c")
        base = wid * TB
        sems = (sem0, sem1)

        sl = TV // NS
        sid = lax.axis_index("s")
        soff = pl.multiple_of(sid * sl, 8)
        pltpu.sync_copy(t_hbm.at[pl.ds(soff, sl)], t_sh.at[pl.ds(soff, sl)])
        plsc.subcore_barrier()

        zero = jnp.zeros((16,), jnp.float32)
        for g in range(NG):
            acc_v[pl.ds(g * 16, 16)] = zero

        def q_descs(q, sem):
            src = t_hbm if q % 2 == 0 else t_sh
            return [
                pltpu.make_async_copy(
                    src.at[idx_v.at[q * QR + r, pl.ds(j * 128, 128)]],
                    vals_v.at[q * QR + r, pl.ds(j * 128, 128)],
                    sem,
                )
                for r in range(QR)
                for j in range(NJ)
            ]

        def reduce_quarter(q):
            def red_body(g, carry):
                off = pl.multiple_of(g * 16, 16)
                s = acc_v[pl.ds(off, 16)]
                for r in range(QR):
                    s = s + vals_v[q * QR + r, pl.ds(off, 16)]
                acc_v[pl.ds(off, 16)] = s
                return carry

            lax.fori_loop(0, NG, red_body, 0)

        def chunk_body(c, carry):
            pltpu.sync_copy(
                idxT_hbm.at[pl.ds(c * LCH, LCH), pl.ds(base, TB)], idx_v
            )
            for d in q_descs(0, sems[0]):
                d.start()
            for d in q_descs(1, sems[1]):
                d.start()
            for q in range(NQ):
                for d in q_descs(q, sems[q % 2]):
                    d.wait()
                if q + 2 < NQ:
                    for d in q_descs(q + 2, sems[q % 2]):
                        d.start()
                reduce_quarter(q)
            return carry

        lax.fori_loop(0, NCH, chunk_body, 0)
        pltpu.sync_copy(acc_v, out_hbm.at[pl.ds(base, TB)])

    return sc_lookup


def kernel(input, input_lengths, emb_weight, lin_weight):
    B, L = input.shape
    tvec = _compute_tvec(emb_weight, lin_weight)
    idxT = _transpose_idx(input)
    out = _make_sc_lookup(B, L, tvec.shape[0])(idxT, tvec)
    return out.reshape(B, 1)


# speedup vs baseline: 4.8797x; 1.0054x over previous
import functools

import jax
import jax.numpy as jnp
from jax import lax
from jax.experimental import pallas as pl
from jax.experimental.pallas import tpu as pltpu
from jax.experimental.pallas import tpu_sc as plsc


_ROWS_BLK = 8192


def _tvec_body(emb_ref, w_ref, t_ref):
    i = pl.program_id(0)
    e = emb_ref[...]
    w = w_ref[...]
    t = lax.dot_general(
        w, e, (((1,), (1,)), ((), ())), preferred_element_type=jnp.float32
    )
    col = lax.broadcasted_iota(jnp.int32, t.shape, 1) + i * _ROWS_BLK
    t = jnp.where(col == 0, 0.0, t)
    t_ref[...] = t.reshape(_ROWS_BLK // 128, 128)


def _compute_tvec(emb_weight, lin_weight):
    V, D = emb_weight.shape
    nblk = pl.cdiv(V, _ROWS_BLK)
    rblk = _ROWS_BLK // 128
    out = pl.pallas_call(
        _tvec_body,
        grid=(nblk,),
        in_specs=[
            pl.BlockSpec((_ROWS_BLK, D), lambda i: (i, 0)),
            pl.BlockSpec((1, D), lambda i: (0, 0)),
        ],
        out_specs=pl.BlockSpec((rblk, 128), lambda i: (i, 0)),
        out_shape=jax.ShapeDtypeStruct((nblk * rblk, 128), jnp.float32),
    )(emb_weight, lin_weight)
    return out.reshape(-1)


_TRB = 512


def _transpose_body(x_ref, o_ref):
    o_ref[...] = x_ref[...].T


def _transpose_idx(x):
    B, L = x.shape
    return pl.pallas_call(
        _transpose_body,
        grid=(B // _TRB,),
        in_specs=[pl.BlockSpec((_TRB, L), lambda i: (i, 0))],
        out_specs=pl.BlockSpec((L, _TRB), lambda i: (0, i)),
        out_shape=jax.ShapeDtypeStruct((L, B), jnp.int32),
    )(x)


@functools.cache
def _make_sc_lookup(B, L, TV):
    info = plsc.get_sparse_core_info()
    NC, NS = info.num_cores, info.num_subcores
    NW = NC * NS
    TB = B // NW
    LCH = 40
    NCH = L // LCH
    QR = 10
    NQ = LCH // QR
    NJ = TB // 128
    NG = TB // 16
    assert B % NW == 0 and L % LCH == 0 and TB % 128 == 0 and LCH % QR == 0

    mesh = plsc.VectorSubcoreMesh(core_axis_name="c", subcore_axis_name="s")

    @functools.partial(
        pl.kernel,
        out_type=jax.ShapeDtypeStruct((B,), jnp.float32),
        mesh=mesh,
        scratch_types=[
            pltpu.VMEM((LCH, TB), jnp.int32),
            pltpu.VMEM((LCH, TB), jnp.float32),
            pltpu.VMEM((TB,), jnp.float32),
            pltpu.VMEM_SHARED((TV,), jnp.float32),
            pltpu.SemaphoreType.DMA,
            pltpu.SemaphoreType.DMA,
        ],
    )
    def sc_lookup(idxT_hbm, t_hbm, out_hbm, idx_v, vals_v, acc_v, t_sh, sem0, sem1):
        wid = lax.axis_index("s") * NC + lax.axis_index("c")
        base = wid * TB
        sems = (sem0, sem1)

        sl = TV // NS
        sid = lax.axis_index("s")
        soff = pl.multiple_of(sid * sl, 8)
        pltpu.sync_copy(t_hbm.at[pl.ds(soff, sl)], t_sh.at[pl.ds(soff, sl)])
        plsc.subcore_barrier()

        zero = jnp.zeros((16,), jnp.float32)
        for g in range(NG):
            acc_v[pl.ds(g * 16, 16)] = zero

        def q_descs(q, sem):
            src = t_hbm if q % 2 == 0 else t_sh
            return [
                pltpu.make_async_copy(
                    src.at[idx_v.at[q * QR + r, pl.ds(j * 128, 128)]],
                    vals_v.at[q * QR + r, pl.ds(j * 128, 128)],
                    sem,
                )
                for r in range(QR)
                for j in range(NJ)
            ]

        def reduce_quarter(q):
            def red_body(g, carry):
                off = pl.multiple_of(g * 16, 16)
                s = acc_v[pl.ds(off, 16)]
                for r in range(QR):
                    s = s + vals_v[q * QR + r, pl.ds(off, 16)]
                acc_v[pl.ds(off, 16)] = s
                return carry

            lax.fori_loop(0, NG, red_body, 0)

        def chunk_body(c, carry):
            pltpu.sync_copy(
                idxT_hbm.at[pl.ds(c * LCH, LCH), pl.ds(base, TB)], idx_v
            )
            for d in q_descs(0, sems[0]):
                d.start()
            for d in q_descs(1, sems[1]):
                d.start()
            for q in range(NQ):
                for d in q_descs(q, sems[q % 2]):
                    d.wait()
                if q + 2 < NQ:
                    for d in q_descs(q + 2, sems[q % 2]):
                        d.start()
                reduce_quarter(q)
            return carry

        lax.fori_loop(0, NCH, chunk_body, 0)
        pltpu.sync_copy(acc_v, out_hbm.at[pl.ds(base, TB)])

    return sc_lookup


def kernel(input, input_lengths, emb_weight, lin_weight):
    B, L = input.shape
    tvec = _compute_tvec(emb_weight, lin_weight)
    idxT = _transpose_idx(input)
    out = _make_sc_lookup(B, L, tvec.shape[0])(idxT, tvec)
    return out.reshape(B, 1)


# speedup vs baseline: 14.2502x; 2.9203x over previous
import functools

import jax
import jax.numpy as jnp
from jax import lax
from jax.experimental import pallas as pl
from jax.experimental.pallas import tpu as pltpu
from jax.experimental.pallas import tpu_sc as plsc


_ROWS_BLK = 8192


def _tvec_body(embt_ref, w_ref, t_ref):
    i = pl.program_id(0)
    e = embt_ref[...]
    w = w_ref[...]
    t = lax.dot_general(
        w, e, (((1,), (0,)), ((), ())), preferred_element_type=jnp.float32
    )
    col = lax.broadcasted_iota(jnp.int32, t.shape, 1) + i * _ROWS_BLK
    t = jnp.where(col == 0, 0.0, t)
    t_ref[...] = t.reshape(_ROWS_BLK // 128, 128)


def _compute_tvec(emb_weight, lin_weight):
    V, D = emb_weight.shape
    nblk = pl.cdiv(V, _ROWS_BLK)
    rblk = _ROWS_BLK // 128
    out = pl.pallas_call(
        _tvec_body,
        grid=(nblk,),
        in_specs=[
            pl.BlockSpec((D, _ROWS_BLK), lambda i: (0, i)),
            pl.BlockSpec((1, D), lambda i: (0, 0)),
        ],
        out_specs=pl.BlockSpec((rblk, 128), lambda i: (i, 0)),
        out_shape=jax.ShapeDtypeStruct((nblk * rblk, 128), jnp.float32),
    )(emb_weight.T, lin_weight)
    return out.reshape(-1)


@functools.cache
def _make_sc_lookup(B, L, TV):
    info = plsc.get_sparse_core_info()
    NC, NS = info.num_cores, info.num_subcores
    NW = NC * NS
    TB = B // NW
    LCH = 40
    NCH = L // LCH
    QR = 10
    NQ = LCH // QR
    NJ = TB // 128
    NG = TB // 16
    assert B % NW == 0 and L % LCH == 0 and TB % 128 == 0 and LCH % QR == 0

    mesh = plsc.VectorSubcoreMesh(core_axis_name="c", subcore_axis_name="s")

    @functools.partial(
        pl.kernel,
        out_type=jax.ShapeDtypeStruct((B,), jnp.float32),
        mesh=mesh,
        scratch_types=[
            pltpu.VMEM((LCH, TB), jnp.int32),
            pltpu.VMEM((LCH, TB), jnp.float32),
            pltpu.VMEM((TB,), jnp.float32),
            pltpu.VMEM_SHARED((TV,), jnp.float32),
            pltpu.SemaphoreType.DMA,
            pltpu.SemaphoreType.DMA,
        ],
    )
    def sc_lookup(idxT_hbm, t_hbm, out_hbm, idx_v, vals_v, acc_v, t_sh, sem0, sem1):
        wid = lax.axis_index("s") * NC + lax.axis_index("c")
        base = wid * TB
        sems = (sem0, sem1)

        sl = TV // NS
        sid = lax.axis_index("s")
        soff = pl.multiple_of(sid * sl, 8)
        pltpu.sync_copy(t_hbm.at[pl.ds(soff, sl)], t_sh.at[pl.ds(soff, sl)])
        plsc.subcore_barrier()

        zero = jnp.zeros((16,), jnp.float32)
        for g in range(NG):
            acc_v[pl.ds(g * 16, 16)] = zero

        def q_descs(q, sem):
            src = t_hbm if q % 2 == 0 else t_sh
            return [
                pltpu.make_async_copy(
                    src.at[idx_v.at[q * QR + r, pl.ds(j * 128, 128)]],
                    vals_v.at[q * QR + r, pl.ds(j * 128, 128)],
                    sem,
                )
                for r in range(QR)
                for j in range(NJ)
            ]

        def reduce_quarter(q):
            def red_body(g, carry):
                off = pl.multiple_of(g * 16, 16)
                s = acc_v[pl.ds(off, 16)]
                for r in range(QR):
                    s = s + vals_v[q * QR + r, pl.ds(off, 16)]
                acc_v[pl.ds(off, 16)] = s
                return carry

            lax.fori_loop(0, NG, red_body, 0)

        def chunk_body(c, carry):
            pltpu.sync_copy(
                idxT_hbm.at[pl.ds(c * LCH, LCH), pl.ds(base, TB)], idx_v
            )
            for d in q_descs(0, sems[0]):
                d.start()
            for d in q_descs(1, sems[1]):
                d.start()
            for q in range(NQ):
                for d in q_descs(q, sems[q % 2]):
                    d.wait()
                if q + 2 < NQ:
                    for d in q_descs(q + 2, sems[q % 2]):
                        d.start()
                reduce_quarter(q)
            return carry

        lax.fori_loop(0, NCH, chunk_body, 0)
        pltpu.sync_copy(acc_v, out_hbm.at[pl.ds(base, TB)])

    return sc_lookup


def kernel(input, input_lengths, emb_weight, lin_weight):
    B, L = input.shape
    tvec = _compute_tvec(emb_weight, lin_weight)
    idxT = input.T
    out = _make_sc_lookup(B, L, tvec.shape[0])(idxT, tvec)
    return out.reshape(B, 1)


# speedup vs baseline: 16.8957x; 1.1856x over previous
import functools

import jax
import jax.numpy as jnp
from jax import lax
from jax.experimental import pallas as pl
from jax.experimental.pallas import tpu as pltpu
from jax.experimental.pallas import tpu_sc as plsc


_ROWS_BLK = 16384


def _tvec_body(embt_ref, w_ref, t_ref):
    i = pl.program_id(0)
    e = embt_ref[...]
    w = w_ref[...]
    t = lax.dot_general(
        w, e, (((1,), (0,)), ((), ())), preferred_element_type=jnp.float32
    )
    col = lax.broadcasted_iota(jnp.int32, t.shape, 1) + i * _ROWS_BLK
    t = jnp.where(col == 0, 0.0, t)
    t_ref[...] = t.reshape(_ROWS_BLK // 128, 128)


def _compute_tvec(emb_weight, lin_weight):
    V, D = emb_weight.shape
    nblk = pl.cdiv(V, _ROWS_BLK)
    rblk = _ROWS_BLK // 128
    out = pl.pallas_call(
        _tvec_body,
        grid=(nblk,),
        in_specs=[
            pl.BlockSpec((D, _ROWS_BLK), lambda i: (0, i)),
            pl.BlockSpec((1, D), lambda i: (0, 0)),
        ],
        out_specs=pl.BlockSpec((rblk, 128), lambda i: (i, 0)),
        out_shape=jax.ShapeDtypeStruct((nblk * rblk, 128), jnp.float32),
    )(emb_weight.T, lin_weight)
    return out.reshape(-1)


@functools.cache
def _make_sc_lookup(B, L, TV):
    info = plsc.get_sparse_core_info()
    NC, NS = info.num_cores, info.num_subcores
    NW = NC * NS
    TB = B // NW
    LCH = 40
    NCH = L // LCH
    QR = 10
    NQ = LCH // QR
    NJ = TB // 128
    NG = TB // 16
    assert B % NW == 0 and L % LCH == 0 and TB % 128 == 0 and LCH % QR == 0

    mesh = plsc.VectorSubcoreMesh(core_axis_name="c", subcore_axis_name="s")

    @functools.partial(
        pl.kernel,
        out_type=jax.ShapeDtypeStruct((B,), jnp.float32),
        mesh=mesh,
        scratch_types=[
            pltpu.VMEM((LCH, TB), jnp.int32),
            pltpu.VMEM((LCH, TB), jnp.float32),
            pltpu.VMEM((TB,), jnp.float32),
            pltpu.VMEM_SHARED((TV,), jnp.float32),
            pltpu.SemaphoreType.DMA,
            pltpu.SemaphoreType.DMA,
        ],
    )
    def sc_lookup(idxT_hbm, t_hbm, out_hbm, idx_v, vals_v, acc_v, t_sh, sem0, sem1):
        wid = lax.axis_index("s") * NC + lax.axis_index("c")
        base = wid * TB
        sems = (sem0, sem1)

        sl = TV // NS
        sid = lax.axis_index("s")
        soff = pl.multiple_of(sid * sl, 8)
        pltpu.sync_copy(t_hbm.at[pl.ds(soff, sl)], t_sh.at[pl.ds(soff, sl)])
        plsc.subcore_barrier()

        zero = jnp.zeros((16,), jnp.float32)
        for g in range(NG):
            acc_v[pl.ds(g * 16, 16)] = zero

        def q_descs(q, sem):
            src = t_hbm if q % 2 == 0 else t_sh
            return [
                pltpu.make_async_copy(
                    src.at[idx_v.at[q * QR + r, pl.ds(j * 128, 128)]],
                    vals_v.at[q * QR + r, pl.ds(j * 128, 128)],
                    sem,
                )
                for r in range(QR)
                for j in range(NJ)
            ]

        def reduce_quarter(q):
            def red_body(g, carry):
                off = pl.multiple_of(g * 16, 16)
                s = acc_v[pl.ds(off, 16)]
                for r in range(QR):
                    s = s + vals_v[q * QR + r, pl.ds(off, 16)]
                acc_v[pl.ds(off, 16)] = s
                return carry

            lax.fori_loop(0, NG, red_body, 0)

        def chunk_body(c, carry):
            pltpu.sync_copy(
                idxT_hbm.at[pl.ds(c * LCH, LCH), pl.ds(base, TB)], idx_v
            )
            for d in q_descs(0, sems[0]):
                d.start()
            for d in q_descs(1, sems[1]):
                d.start()
            for q in range(NQ):
                for d in q_descs(q, sems[q % 2]):
                    d.wait()
                if q + 2 < NQ:
                    for d in q_descs(q + 2, sems[q % 2]):
                        d.start()
                reduce_quarter(q)
            return carry

        lax.fori_loop(0, NCH, chunk_body, 0)
        pltpu.sync_copy(acc_v, out_hbm.at[pl.ds(base, TB)])

    return sc_lookup


def kernel(input, input_lengths, emb_weight, lin_weight):
    B, L = input.shape
    tvec = _compute_tvec(emb_weight, lin_weight)
    idxT = input.T
    out = _make_sc_lookup(B, L, tvec.shape[0])(idxT, tvec)
    return out.reshape(B, 1)
